# Initial kernel scaffold; baseline (speedup 1.0000x reference)
import jax
import jax.numpy as jnp
from jax.experimental import pallas as pl


def kernel(feat_stage3, feat_stage4, feat_stage5, intrinsics, extrinsics, params):
    raise NotImplementedError("write your pallas kernel here")


# speedup vs baseline: 1.2636x; 1.2636x over previous
import functools

import jax
import jax.numpy as jnp
import numpy as np
from jax.experimental import pallas as pl
from jax.experimental.pallas import tpu as pltpu

_IMG_CH = {'stage3': 128, 'stage4': 256, 'stage5': 512}
_SHAPES = {'stage3': (64, 176), 'stage4': (32, 88), 'stage5': (16, 44)}
_B = 6
_BEV_H = 128
_BEV_W = 128
_DEPTH_CH = 64
_VOXEL = 0.8
_NCELL = _BEV_H * _BEV_W
_NPAD = 16384
_DEPTH_BINS = jnp.exp(
    jnp.linspace(float(np.log(1.0)), float(np.log(35.0)), _DEPTH_CH)
).astype(jnp.float32)


def _conv(x, w, b=None, padding=0, groups=1):
    out = jax.lax.conv_general_dilated(
        x, w, (1, 1), [(padding, padding), (padding, padding)],
        dimension_numbers=('NCHW', 'OIHW', 'NCHW'), feature_group_count=groups)
    if b is not None:
        out = out + b[None, :, None, None]
    return out


def _bn(x, g, b):
    return x * (g / np.float32(np.sqrt(1.0 + 1e-5)))[None, :, None, None] + b[None, :, None, None]


def _pixel_grid(H, W):
    x = jnp.linspace(0.0, W - 1.0, W)
    y = jnp.linspace(0.0, H - 1.0, H)
    yy, xx = jnp.meshgrid(y, x, indexing='ij')
    return jnp.stack([xx, yy, jnp.ones_like(xx)], axis=-1).reshape(-1, 3).T


def _premult_body(w_ref, w1_ref, v_ref):
    w = w_ref[0]
    w1 = w1_ref[...]
    v_ref[0] = jax.lax.dot_general(
        w, w1, (((0,), (1,)), ((), ())), preferred_element_type=jnp.float32)


def _premult(weighted, w1s, hwb):
    b, c, hw = weighted.shape
    grid = (b, hw // hwb)
    return pl.pallas_call(
        _premult_body,
        grid=grid,
        in_specs=[
            pl.BlockSpec((1, 128, hwb), lambda i, j: (i, 0, j)),
            pl.BlockSpec((256, 128), lambda i, j: (0, 0)),
        ],
        out_specs=pl.BlockSpec((1, hwb, 256), lambda i, j: (i, j, 0)),
        out_shape=jax.ShapeDtypeStruct((b, hw, 256), jnp.float32),
    )(weighted, w1s)


def _fusion_body(z_ref, g_ref, bb_ref, w2_ref, b2_ref, out_ref):
    z = z_ref[0]
    act = jnp.maximum(z * g_ref[...] + bb_ref[...], 0.0)
    out = jax.lax.dot_general(
        w2_ref[...], act, (((1,), (1,)), ((), ())),
        preferred_element_type=jnp.float32)
    out_ref[0] = out + b2_ref[...]


def _fusion(z, g, bb, w2, b2, cb=2048):
    b, ncell, _ = z.shape
    grid = (b, ncell // cb)
    return pl.pallas_call(
        _fusion_body,
        grid=grid,
        in_specs=[
            pl.BlockSpec((1, cb, 256), lambda i, j: (i, j, 0)),
            pl.BlockSpec((1, 256), lambda i, j: (0, 0)),
            pl.BlockSpec((1, 256), lambda i, j: (0, 0)),
            pl.BlockSpec((128, 256), lambda i, j: (0, 0)),
            pl.BlockSpec((128, 1), lambda i, j: (0, 0)),
        ],
        out_specs=pl.BlockSpec((1, 128, cb), lambda i, j: (i, 0, j)),
        out_shape=jax.ShapeDtypeStruct((b, 128, ncell), jnp.float32),
    )(z, g, bb, w2, b2)


def _scatter(vals, idx):
    def one(v, i):
        return jnp.zeros((_NCELL, 256), jnp.float32).at[i].add(v)
    return jax.vmap(one)(vals, idx)


def kernel(feat_stage3, feat_stage4, feat_stage5, intrinsics, extrinsics, params):
    feats = {'stage3': feat_stage3, 'stage4': feat_stage4, 'stage5': feat_stage5}
    K_inv = jnp.linalg.inv(intrinsics)
    T = extrinsics.reshape(_B, 4, 4)

    fp = params['fusion']
    W1 = fp['fu1_w'][:, :, 0, 0]

    vals_list, idx_list = [], []
    for si, scale in enumerate(_IMG_CH):
        f = feats[scale]
        p = params[scale]
        _, _, H, W = f.shape
        hw = H * W
        grid = _pixel_grid(H, W)

        h = jax.nn.relu(_bn(_conv(f, p['fr1_w']), p['fr_bn1_g'], p['fr_bn1_b']))
        reduced = jax.nn.relu(_bn(_conv(h, p['fr2_w'], padding=1, groups=8),
                                  p['fr_bn2_g'], p['fr_bn2_b']))
        d = jax.nn.relu(_bn(_conv(f, p['dn1_w']), p['dn_bn_g'], p['dn_bn_b']))
        depth_logits = _conv(d, p['dn2_w'], p['dn2_b'])
        depth_probs = jax.nn.softmax(depth_logits * 10.0, axis=1)
        depth_map = (depth_probs * _DEPTH_BINS[None, :, None, None]).sum(axis=1)
        c = jax.nn.relu(_bn(_conv(jnp.concatenate([depth_logits, reduced], axis=1),
                                  p['cn1_w'], padding=1), p['cn_bn_g'], p['cn_bn_b']))
        confidence = jax.nn.sigmoid(_conv(c, p['cn2_w'], p['cn2_b']))

        depth_flat = depth_map.reshape(_B, 1, hw)
        cam_pts = depth_flat * jnp.matmul(K_inv, grid[None])
        cam_pts_h = jnp.concatenate([cam_pts, jnp.ones_like(cam_pts[:, :1])], axis=1)
        ego = jnp.matmul(T, cam_pts_h)[:, :3]
        bev_x = (ego[:, 0] / _VOXEL + _BEV_W // 2).astype(jnp.int32)
        bev_y = (ego[:, 1] / _VOXEL + _BEV_H // 2).astype(jnp.int32)
        valid = (bev_x >= 0) & (bev_x < _BEV_W) & (bev_y >= 0) & (bev_y < _BEV_H)
        idx = jnp.where(valid, bev_y * _BEV_W + bev_x, 0)

        weighted = reduced.reshape(_B, 128, hw) * confidence.reshape(_B, 1, hw)
        weighted = jnp.where(valid[:, None, :], weighted, 0.0)

        w1s = W1[:, si * 128:(si + 1) * 128]
        hwb = 1408 if hw == 11264 else hw
        vals_list.append(_premult(weighted, w1s, hwb))
        idx_list.append(idx)

    vals = jnp.concatenate(vals_list, axis=1)
    idx = jnp.concatenate(idx_list, axis=1)
    npts = vals.shape[1]
    vals = jnp.pad(vals, ((0, 0), (0, _NPAD - npts), (0, 0)))
    idx = jnp.pad(idx, ((0, 0), (0, _NPAD - npts)))

    z = _scatter(vals, idx)

    g1 = (fp['fu_bn_g'] / np.float32(np.sqrt(1.0 + 1e-5))).reshape(1, 256)
    b1 = fp['fu_bn_b'].reshape(1, 256)
    w2 = fp['fu2_w'][:, :, 0, 0]
    b2 = fp['fu2_b'].reshape(128, 1)
    out = _fusion(z, g1, b1, w2, b2)
    return out.reshape(_B, 128, _BEV_H, _BEV_W)


# speedup vs baseline: 1.4679x; 1.1617x over previous
import jax
import jax.numpy as jnp
import numpy as np
from jax.experimental import pallas as pl

_IMG_CH = {'stage3': 128, 'stage4': 256, 'stage5': 512}
_B = 6
_BEV_H = 128
_BEV_W = 128
_DEPTH_CH = 64
_VOXEL = 0.8
_NCELL = _BEV_H * _BEV_W
_DEPTH_BINS = jnp.exp(
    jnp.linspace(float(np.log(1.0)), float(np.log(35.0)), _DEPTH_CH)
).astype(jnp.float32)


def _conv(x, w, b=None, padding=0, groups=1):
    out = jax.lax.conv_general_dilated(
        x, w, (1, 1), [(padding, padding), (padding, padding)],
        dimension_numbers=('NCHW', 'OIHW', 'NCHW'), feature_group_count=groups)
    if b is not None:
        out = out + b[None, :, None, None]
    return out


def _bn(x, g, b):
    return x * (g / np.float32(np.sqrt(1.0 + 1e-5)))[None, :, None, None] + b[None, :, None, None]


def _pixel_grid(H, W):
    x = jnp.linspace(0.0, W - 1.0, W)
    y = jnp.linspace(0.0, H - 1.0, H)
    yy, xx = jnp.meshgrid(y, x, indexing='ij')
    return jnp.stack([xx, yy, jnp.ones_like(xx)], axis=-1).reshape(-1, 3).T


def _premult_body(w_ref, w1_ref, v_ref):
    v_ref[0] = jax.lax.dot_general(
        w_ref[0], w1_ref[...], (((0,), (1,)), ((), ())),
        preferred_element_type=jnp.float32)


def _premult(weighted, w1s, hwb):
    b, _, hw = weighted.shape
    grid = (b, hw // hwb)
    return pl.pallas_call(
        _premult_body,
        grid=grid,
        in_specs=[
            pl.BlockSpec((1, 128, hwb), lambda i, j: (i, 0, j)),
            pl.BlockSpec((256, 128), lambda i, j: (0, 0)),
        ],
        out_specs=pl.BlockSpec((1, hwb, 256), lambda i, j: (i, j, 0)),
        out_shape=jax.ShapeDtypeStruct((b, hw, 256), jnp.float32),
    )(weighted, w1s)


def _fusion_body(z_ref, g_ref, bb_ref, w2_ref, b2_ref, out_ref):
    z = z_ref[0]
    act = jnp.maximum(z * g_ref[...] + bb_ref[...], 0.0)
    out = jax.lax.dot_general(
        w2_ref[...], act, (((1,), (1,)), ((), ())),
        preferred_element_type=jnp.float32)
    out_ref[0] = out + b2_ref[...]


def _fusion(z, g, bb, w2, b2, cb=2048):
    b, ncell, _ = z.shape
    grid = (b, ncell // cb)
    return pl.pallas_call(
        _fusion_body,
        grid=grid,
        in_specs=[
            pl.BlockSpec((1, cb, 256), lambda i, j: (i, j, 0)),
            pl.BlockSpec((1, 256), lambda i, j: (0, 0)),
            pl.BlockSpec((1, 256), lambda i, j: (0, 0)),
            pl.BlockSpec((128, 256), lambda i, j: (0, 0)),
            pl.BlockSpec((128, 1), lambda i, j: (0, 0)),
        ],
        out_specs=pl.BlockSpec((1, 128, cb), lambda i, j: (i, 0, j)),
        out_shape=jax.ShapeDtypeStruct((b, 128, ncell), jnp.float32),
    )(z, g, bb, w2, b2)


def kernel(feat_stage3, feat_stage4, feat_stage5, intrinsics, extrinsics, params):
    feats = {'stage3': feat_stage3, 'stage4': feat_stage4, 'stage5': feat_stage5}
    K_inv = jnp.linalg.inv(intrinsics)
    T = extrinsics.reshape(_B, 4, 4)

    fp = params['fusion']
    W1 = fp['fu1_w'][:, :, 0, 0]

    vals_list, idx_list = [], []
    for si, scale in enumerate(_IMG_CH):
        f = feats[scale]
        p = params[scale]
        _, _, H, W = f.shape
        hw = H * W
        grid = _pixel_grid(H, W)

        h = jax.nn.relu(_bn(_conv(f, p['fr1_w']), p['fr_bn1_g'], p['fr_bn1_b']))
        reduced = jax.nn.relu(_bn(_conv(h, p['fr2_w'], padding=1, groups=8),
                                  p['fr_bn2_g'], p['fr_bn2_b']))
        d = jax.nn.relu(_bn(_conv(f, p['dn1_w']), p['dn_bn_g'], p['dn_bn_b']))
        depth_logits = _conv(d, p['dn2_w'], p['dn2_b'])
        depth_probs = jax.nn.softmax(depth_logits * 10.0, axis=1)
        depth_map = (depth_probs * _DEPTH_BINS[None, :, None, None]).sum(axis=1)
        c = jax.nn.relu(_bn(_conv(jnp.concatenate([depth_logits, reduced], axis=1),
                                  p['cn1_w'], padding=1), p['cn_bn_g'], p['cn_bn_b']))
        confidence = jax.nn.sigmoid(_conv(c, p['cn2_w'], p['cn2_b']))

        depth_flat = depth_map.reshape(_B, 1, hw)
        cam_pts = depth_flat * jnp.matmul(K_inv, grid[None])
        cam_pts_h = jnp.concatenate([cam_pts, jnp.ones_like(cam_pts[:, :1])], axis=1)
        ego = jnp.matmul(T, cam_pts_h)[:, :3]
        bev_x = (ego[:, 0] / _VOXEL + _BEV_W // 2).astype(jnp.int32)
        bev_y = (ego[:, 1] / _VOXEL + _BEV_H // 2).astype(jnp.int32)
        valid = (bev_x >= 0) & (bev_x < _BEV_W) & (bev_y >= 0) & (bev_y < _BEV_H)
        idx = jnp.where(valid, bev_y * _BEV_W + bev_x, 0)

        weighted = reduced.reshape(_B, 128, hw) * confidence.reshape(_B, 1, hw)
        weighted = jnp.where(valid[:, None, :], weighted, 0.0)

        w1s = W1[:, si * 128:(si + 1) * 128]
        hwb = 1408 if hw == 11264 else hw
        vals_list.append(_premult(weighted, w1s, hwb))
        idx_list.append(idx)

    def _scatter_one(v3, i3, v4, i4, v5, i5):
        z = jnp.zeros((_NCELL, 256), jnp.float32)
        z = z.at[i3].add(v3)
        z = z.at[i4].add(v4)
        z = z.at[i5].add(v5)
        return z

    z = jax.vmap(_scatter_one)(vals_list[0], idx_list[0],
                               vals_list[1], idx_list[1],
                               vals_list[2], idx_list[2])

    g1 = (fp['fu_bn_g'] / np.float32(np.sqrt(1.0 + 1e-5))).reshape(1, 256)
    b1 = fp['fu_bn_b'].reshape(1, 256)
    w2 = fp['fu2_w'][:, :, 0, 0]
    b2 = fp['fu2_b'].reshape(128, 1)
    out = _fusion(z, g1, b1, w2, b2)
    return out.reshape(_B, 128, _BEV_H, _BEV_W)
